# Initial kernel scaffold; baseline (speedup 1.0000x reference)
import jax
import jax.numpy as jnp
from jax.experimental import pallas as pl


def kernel(x, edge_index, weight, W1, att_src1, att_dst1, b1, W2, att_src2, att_dst2, b2):
    raise NotImplementedError("write your pallas kernel here")


# speedup vs baseline: 14.8120x; 14.8120x over previous
import functools

import jax
import jax.numpy as jnp
from jax import lax
from jax.experimental import pallas as pl
from jax.experimental.pallas import tpu as pltpu
from jax.experimental.pallas import tpu_sc as plsc

_N = 10000
_NP = 10240
_E = 320000
_TILES = 32
_CH = 80
_K = 128
_EP = _TILES * _CH * _K
_RPT = _NP // 16
_LANES = 16


def _mm_alpha_body(x_ref, w_ref, as_ref, ad_ref, h_ref, s_ref, d_ref):
    h = jnp.dot(x_ref[...], w_ref[...], preferred_element_type=jnp.float32)
    h_ref[...] = h
    s_ref[...] = jnp.sum(h * as_ref[...], axis=1)
    d_ref[...] = jnp.sum(h * ad_ref[...], axis=1)


def _mm_alpha(x, W, a_src, a_dst):
    H = W.shape[1]
    BM = 1024
    return pl.pallas_call(
        _mm_alpha_body,
        grid=(_NP // BM,),
        in_specs=[
            pl.BlockSpec((BM, W.shape[0]), lambda i: (i, 0)),
            pl.BlockSpec((W.shape[0], H), lambda i: (0, 0)),
            pl.BlockSpec((1, H), lambda i: (0, 0)),
            pl.BlockSpec((1, H), lambda i: (0, 0)),
        ],
        out_specs=[
            pl.BlockSpec((BM, H), lambda i: (i, 0)),
            pl.BlockSpec((BM,), lambda i: (i,)),
            pl.BlockSpec((BM,), lambda i: (i,)),
        ],
        out_shape=[
            jax.ShapeDtypeStruct((_NP, H), jnp.float32),
            jax.ShapeDtypeStruct((_NP,), jnp.float32),
            jax.ShapeDtypeStruct((_NP,), jnp.float32),
        ],
    )(x, W, a_src.reshape(1, H), a_dst.reshape(1, H))


def _sc_edge_body(D, h_hbm, asrc_hbm, adst_hbm, src_hbm, dst_hbm,
                  s_out, den_out,
                  src_v, dst_v, av_v, bv_v, ex_v, rows_v, zden_v,
                  s_sh, den_sh, sem, sem_h):
    cid = lax.axis_index("c")
    sid = lax.axis_index("s")
    t = cid * 16 + sid
    zeros16 = jnp.zeros((_LANES,), jnp.float32)

    def _zrow(i, carry):
        for g in range(D // _LANES):
            rows_v[i, pl.ds(g * _LANES, _LANES)] = zeros16
        return carry
    lax.fori_loop(0, _K, _zrow, 0)

    def _zden(i, carry):
        zden_v[pl.ds(i * _LANES, _LANES)] = zeros16
        return carry
    lax.fori_loop(0, _RPT // _LANES, _zden, 0)

    base = sid * _RPT
    pltpu.sync_copy(zden_v, den_sh.at[pl.ds(base, _RPT)])
    for j in range(_RPT // _K):
        pltpu.sync_copy(rows_v, s_sh.at[pl.ds(base + j * _K, _K)])

    pltpu.sync_copy(src_hbm.at[pl.ds(t * _CH, _CH)], src_v)
    pltpu.sync_copy(dst_hbm.at[pl.ds(t * _CH, _CH)], dst_v)

    plsc.subcore_barrier()

    def _chunk(c, carry):
        hcopy = pltpu.async_copy(h_hbm.at[src_v.at[c]], rows_v, sem_h)
        pltpu.async_copy(asrc_hbm.at[src_v.at[c]], av_v, sem).wait()
        pltpu.async_copy(adst_hbm.at[dst_v.at[c]], bv_v, sem).wait()

        for g in range(_K // _LANES):
            sl = pl.ds(g * _LANES, _LANES)
            al = av_v[sl] + bv_v[sl]
            al = jnp.where(al >= 0.0, al, al * jnp.float32(0.2))
            ex_v[sl] = jnp.exp(al)

        pltpu.sync_copy(ex_v, den_sh.at[dst_v.at[c]], add=True)
        hcopy.wait()

        def _scale(e, carry2):
            exs = plsc.load_gather(ex_v, [jnp.full((_LANES,), e, jnp.int32)])
            for g in range(D // _LANES):
                sl = pl.ds(g * _LANES, _LANES)
                rows_v[e, sl] = rows_v[e, sl] * exs
            return carry2
        lax.fori_loop(0, _K, _scale, 0)

        pltpu.sync_copy(rows_v, s_sh.at[dst_v.at[c]], add=True)
        return carry
    lax.fori_loop(0, _CH, _chunk, 0)

    plsc.subcore_barrier()

    for j in range(_RPT // _K):
        pltpu.sync_copy(s_sh.at[pl.ds(base + j * _K, _K)], rows_v)
        pltpu.sync_copy(rows_v, s_out.at[cid, pl.ds(base + j * _K, _K)])
    pltpu.sync_copy(den_sh.at[pl.ds(base, _RPT)], zden_v)
    pltpu.sync_copy(zden_v, den_out.at[cid, pl.ds(base, _RPT)])


def _sc_edge(h, asrc, adst, srcp, dstp):
    D = h.shape[1]
    mesh = plsc.VectorSubcoreMesh(core_axis_name="c", subcore_axis_name="s",
                                  num_cores=2, num_subcores=16)
    f = pl.kernel(
        functools.partial(_sc_edge_body, D),
        out_type=[
            jax.ShapeDtypeStruct((2, _NP, D), jnp.float32),
            jax.ShapeDtypeStruct((2, _NP), jnp.float32),
        ],
        mesh=mesh,
        compiler_params=pltpu.CompilerParams(needs_layout_passes=False),
        scratch_types=[
            pltpu.VMEM((_CH, _K), jnp.int32),
            pltpu.VMEM((_CH, _K), jnp.int32),
            pltpu.VMEM((_K,), jnp.float32),
            pltpu.VMEM((_K,), jnp.float32),
            pltpu.VMEM((_K,), jnp.float32),
            pltpu.VMEM((_K, D), jnp.float32),
            pltpu.VMEM((_RPT,), jnp.float32),
            pltpu.VMEM_SHARED((_NP, D), jnp.float32),
            pltpu.VMEM_SHARED((_NP,), jnp.float32),
            pltpu.SemaphoreType.DMA,
            pltpu.SemaphoreType.DMA,
        ],
    )
    return f(h, asrc, adst, srcp, dstp)


def _mid_body(N_total, s0_ref, s1_ref, d0_ref, d1_ref, b_ref, w_ref,
              as_ref, ad_ref, h_ref, sa_ref, da_ref):
    i = pl.program_id(0)
    BM = s0_ref.shape[0]
    den = d0_ref[...] + d1_ref[...] + jnp.float32(1e-16)
    h1 = (s0_ref[...] + s1_ref[...]) / den[:, None] + b_ref[...]
    h1 = jnp.maximum(h1, 0.0)
    ridx = lax.broadcasted_iota(jnp.int32, (BM, 1), 0) + i * BM
    h1 = jnp.where(ridx < N_total, h1, 0.0)
    h2 = jnp.dot(h1, w_ref[...], preferred_element_type=jnp.float32)
    h_ref[...] = h2
    sa_ref[...] = jnp.sum(h2 * as_ref[...], axis=1)
    da_ref[...] = jnp.sum(h2 * ad_ref[...], axis=1)


def _mid(s0, s1, d0, d1, b1, W2, a_src2, a_dst2):
    H = s0.shape[1]
    D = W2.shape[1]
    BM = 1024
    return pl.pallas_call(
        functools.partial(_mid_body, _N),
        grid=(_NP // BM,),
        in_specs=[
            pl.BlockSpec((BM, H), lambda i: (i, 0)),
            pl.BlockSpec((BM, H), lambda i: (i, 0)),
            pl.BlockSpec((BM,), lambda i: (i,)),
            pl.BlockSpec((BM,), lambda i: (i,)),
            pl.BlockSpec((1, H), lambda i: (0, 0)),
            pl.BlockSpec((H, D), lambda i: (0, 0)),
            pl.BlockSpec((1, D), lambda i: (0, 0)),
            pl.BlockSpec((1, D), lambda i: (0, 0)),
        ],
        out_specs=[
            pl.BlockSpec((BM, D), lambda i: (i, 0)),
            pl.BlockSpec((BM,), lambda i: (i,)),
            pl.BlockSpec((BM,), lambda i: (i,)),
        ],
        out_shape=[
            jax.ShapeDtypeStruct((_NP, D), jnp.float32),
            jax.ShapeDtypeStruct((_NP,), jnp.float32),
            jax.ShapeDtypeStruct((_NP,), jnp.float32),
        ],
    )(s0, s1, d0, d1, b1.reshape(1, H), W2,
      a_src2.reshape(1, D), a_dst2.reshape(1, D))


def _final_body(s0_ref, s1_ref, d0_ref, d1_ref, b_ref, o_ref):
    den = d0_ref[...] + d1_ref[...] + jnp.float32(1e-16)
    o = (s0_ref[...] + s1_ref[...]) / den[:, None] + b_ref[...]
    o_ref[...] = jnp.maximum(o, 0.0)


def _final(s0, s1, d0, d1, b2):
    D = s0.shape[1]
    BM = 1024
    return pl.pallas_call(
        _final_body,
        grid=(pl.cdiv(_N, BM),),
        in_specs=[
            pl.BlockSpec((BM, D), lambda i: (i, 0)),
            pl.BlockSpec((BM, D), lambda i: (i, 0)),
            pl.BlockSpec((BM,), lambda i: (i,)),
            pl.BlockSpec((BM,), lambda i: (i,)),
            pl.BlockSpec((1, D), lambda i: (0, 0)),
        ],
        out_specs=pl.BlockSpec((BM, D), lambda i: (i, 0)),
        out_shape=jax.ShapeDtypeStruct((_N, D), jnp.float32),
    )(s0, s1, d0, d1, b2.reshape(1, D))


def kernel(x, edge_index, weight, W1, att_src1, att_dst1, b1,
           W2, att_src2, att_dst2, b2):
    del weight
    pad_e = jnp.full((_EP - _E,), _NP - 1, jnp.int32)
    srcp = jnp.concatenate([edge_index[0], pad_e]).reshape(_TILES * _CH, _K)
    dstp = jnp.concatenate([edge_index[1], pad_e]).reshape(_TILES * _CH, _K)
    x_p = jnp.pad(x, ((0, _NP - _N), (0, 0)))

    h1, as1, ad1 = _mm_alpha(x_p, W1, att_src1, att_dst1)
    S1, den1 = _sc_edge(h1, as1, ad1, srcp, dstp)
    W2p = jnp.pad(W2, ((0, 0), (0, 128 - W2.shape[1])))
    as2p = jnp.pad(att_src2, (0, 128 - att_src2.shape[0]))
    ad2p = jnp.pad(att_dst2, (0, 128 - att_dst2.shape[0]))
    h2, as2, ad2 = _mid(S1[0], S1[1], den1[0], den1[1], b1, W2p, as2p, ad2p)
    S2, den2 = _sc_edge(h2, as2, ad2, srcp, dstp)
    D2 = W2.shape[1]
    return _final(S2[0][:, :D2], S2[1][:, :D2], den2[0], den2[1], b2)
